# baseline (speedup 1.0000x reference)
import functools

import jax
import jax.numpy as jnp
from jax import lax
from jax.experimental import pallas as pl
from jax.experimental.pallas import tpu as pltpu
import jax.experimental.pallas.tpu_sc as plsc

N_NODES = 10000
N_PAD = 10240
N_EDGES = 320000
D_IN = 128
D_HID = 256
D_OUT = 128
DEG_W = 16

NC = 2
NS = 16
E_PER_SC = N_EDGES // NC
E_PER_TILE = E_PER_SC // NS
CHUNK = 40
N_CHUNKS = E_PER_TILE // CHUNK
ROWS_PER_TILE = N_PAD // NS

ROW_BLK = 640


def _proj_body(x_ref, win_ref, bin_ref, wl_ref, wr_ref, bl_ref,
               g_ref, r_ref):
    x = x_ref[...]
    h = jax.lax.dot_general(x, win_ref[...], (((1,), (1,)), ((), ())),
                            preferred_element_type=jnp.float32)
    h = jnp.maximum(h + bin_ref[...], 0.0)
    g_ref[...] = jax.lax.dot_general(h, wl_ref[...], (((1,), (1,)), ((), ())),
                                     preferred_element_type=jnp.float32)
    r_ref[...] = jax.lax.dot_general(h, wr_ref[...], (((1,), (1,)), ((), ())),
                                     preferred_element_type=jnp.float32) + bl_ref[...]


def _final_body(p0_ref, p1_ref, d0_ref, d1_ref, r_ref, out_ref):
    s = p0_ref[...] + p1_ref[...]
    deg = d0_ref[:, 0:1] + d1_ref[:, 0:1]
    mean = s / jnp.maximum(deg, 1.0)
    o = mean + r_ref[...]
    m = jnp.max(o, axis=1, keepdims=True)
    lse = jnp.log(jnp.sum(jnp.exp(o - m), axis=1, keepdims=True))
    out_ref[...] = o - m - lse


def _sc_edge_agg(g, src, dst, zeros_p, zeros_d):
    mesh = plsc.VectorSubcoreMesh(core_axis_name="c", subcore_axis_name="s")

    @functools.partial(
        pl.kernel,
        out_type=(
            jax.ShapeDtypeStruct((NC * N_PAD, D_OUT), jnp.float32),
            jax.ShapeDtypeStruct((NC * N_PAD, DEG_W), jnp.float32),
        ),
        mesh=mesh,
        compiler_params=pltpu.CompilerParams(use_tc_tiling_on_sc=False),
        scratch_types=[
            pltpu.VMEM((N_CHUNKS, CHUNK), jnp.int32),
            pltpu.VMEM((N_CHUNKS, CHUNK), jnp.int32),
            pltpu.VMEM((3, CHUNK, D_OUT), jnp.float32),
            pltpu.VMEM((CHUNK, DEG_W), jnp.float32),
            pltpu.VMEM_SHARED((N_PAD, D_OUT), jnp.float32),
            pltpu.VMEM_SHARED((N_PAD, DEG_W), jnp.float32),
            pltpu.SemaphoreType.DMA,
            pltpu.SemaphoreType.DMA,
            pltpu.SemaphoreType.DMA,
            pltpu.SemaphoreType.DMA,
        ],
    )
    def edge_agg(g_hbm, src_hbm, dst_hbm, zp_hbm, zd_hbm, out_hbm, deg_hbm,
                 src_v, dst_v, rows_v, ones_v, acc_sh, dega_sh,
                 sem0, sem1, sem2, osem):
        c = lax.axis_index("c")
        s = lax.axis_index("s")
        sems = (sem0, sem1, sem2)

        stripe = pl.ds(s * ROWS_PER_TILE, ROWS_PER_TILE)
        pltpu.sync_copy(zp_hbm.at[stripe], acc_sh.at[stripe])
        pltpu.sync_copy(zd_hbm.at[stripe], dega_sh.at[stripe])
        pltpu.sync_copy(src_hbm.at[c, s], src_v)
        pltpu.sync_copy(dst_hbm.at[c, s], dst_v)

        lane = lax.broadcasted_iota(jnp.int32, (DEG_W,), 0)
        one_vec = jnp.where(lane == 0, 1.0, 0.0).astype(jnp.float32)

        def fill(r, carry):
            ones_v[r, :] = one_vec
            return carry

        lax.fori_loop(0, CHUNK, fill, 0)
        plsc.subcore_barrier()

        def gather(i, b):
            pltpu.async_copy(g_hbm.at[src_v.at[i]], rows_v.at[b], sems[b])

        def gwait(i, b):
            pltpu.make_async_copy(g_hbm.at[src_v.at[i]], rows_v.at[b],
                                  sems[b]).wait()

        def step(i, b):
            gwait(i, b)

            @pl.when(i + 2 < N_CHUNKS)
            def _():
                gather(i + 2, (b + 2) % 3)

            pltpu.sync_copy(rows_v.at[b], acc_sh.at[dst_v.at[i]], add=True)
            pltpu.async_copy(ones_v, dega_sh.at[dst_v.at[i]], osem, add=True)

        gather(0, 0)
        gather(1, 1)

        def body(j, carry):
            i0 = j * 3
            step(i0, 0)
            step(i0 + 1, 1)
            step(i0 + 2, 2)
            return carry

        n_triples = N_CHUNKS // 3
        lax.fori_loop(0, n_triples, body, 0)
        for i in range(n_triples * 3, N_CHUNKS):
            step(i, i % 3)

        def drain(i, carry):
            pltpu.make_async_copy(ones_v, dega_sh.at[dst_v.at[i]],
                                  osem).wait()
            return carry

        lax.fori_loop(0, N_CHUNKS, drain, 0)
        plsc.subcore_barrier()

        out_stripe = pl.ds(c * N_PAD + s * ROWS_PER_TILE, ROWS_PER_TILE)
        pltpu.sync_copy(acc_sh.at[stripe], out_hbm.at[out_stripe])
        pltpu.sync_copy(dega_sh.at[stripe], deg_hbm.at[out_stripe])

    return edge_agg(g, src, dst, zeros_p, zeros_d)


def kernel(x, edge_index, W_in, b_in, W_l, b_l, W_r):
    src = edge_index[0].reshape(NC, NS, N_CHUNKS, CHUNK)
    dst = edge_index[1].reshape(NC, NS, N_CHUNKS, CHUNK)
    x_pad = jnp.pad(x, ((0, N_PAD - N_NODES), (0, 0)))
    b_in2 = b_in.reshape(1, D_HID)
    b_l2 = b_l.reshape(1, D_OUT)

    n_blocks = N_PAD // ROW_BLK
    g, r = pl.pallas_call(
        _proj_body,
        grid=(n_blocks,),
        in_specs=[
            pl.BlockSpec((ROW_BLK, D_IN), lambda i: (i, 0)),
            pl.BlockSpec((D_HID, D_IN), lambda i: (0, 0)),
            pl.BlockSpec((1, D_HID), lambda i: (0, 0)),
            pl.BlockSpec((D_OUT, D_HID), lambda i: (0, 0)),
            pl.BlockSpec((D_OUT, D_HID), lambda i: (0, 0)),
            pl.BlockSpec((1, D_OUT), lambda i: (0, 0)),
        ],
        out_specs=[
            pl.BlockSpec((ROW_BLK, D_OUT), lambda i: (i, 0)),
            pl.BlockSpec((ROW_BLK, D_OUT), lambda i: (i, 0)),
        ],
        out_shape=[
            jax.ShapeDtypeStruct((N_PAD, D_OUT), jnp.float32),
            jax.ShapeDtypeStruct((N_PAD, D_OUT), jnp.float32),
        ],
    )(x_pad, W_in, b_in2, W_l, W_r, b_l2)

    zeros_p = jnp.zeros((N_PAD, D_OUT), jnp.float32)
    zeros_d = jnp.zeros((N_PAD, DEG_W), jnp.float32)
    parts, degs = _sc_edge_agg(g, src, dst, zeros_p, zeros_d)
    p0 = parts[:N_PAD]
    p1 = parts[N_PAD:]
    d0 = degs[:N_PAD]
    d1 = degs[N_PAD:]

    out = pl.pallas_call(
        _final_body,
        grid=(n_blocks,),
        in_specs=[
            pl.BlockSpec((ROW_BLK, D_OUT), lambda i: (i, 0)),
            pl.BlockSpec((ROW_BLK, D_OUT), lambda i: (i, 0)),
            pl.BlockSpec((ROW_BLK, DEG_W), lambda i: (i, 0)),
            pl.BlockSpec((ROW_BLK, DEG_W), lambda i: (i, 0)),
            pl.BlockSpec((ROW_BLK, D_OUT), lambda i: (i, 0)),
        ],
        out_specs=pl.BlockSpec((ROW_BLK, D_OUT), lambda i: (i, 0)),
        out_shape=jax.ShapeDtypeStruct((N_PAD, D_OUT), jnp.float32),
    )(p0, p1, d0, d1, r)
    return out[:N_NODES]


# speedup vs baseline: 1.1050x; 1.1050x over previous
import functools

import jax
import jax.numpy as jnp
from jax import lax
from jax.experimental import pallas as pl
from jax.experimental.pallas import tpu as pltpu
import jax.experimental.pallas.tpu_sc as plsc

N_NODES = 10000
N_PAD = 10240
N_EDGES = 320000
D_IN = 128
D_HID = 256
D_OUT = 128
DEG_W = 16

NC = 2
NS = 16
E_PER_SC = N_EDGES // NC
E_PER_TILE = E_PER_SC // NS
CHUNK = 40
N_CHUNKS = E_PER_TILE // CHUNK
ROWS_PER_TILE = N_PAD // NS

ROW_BLK = 1024


def _proj_body(x_ref, win_ref, bin_ref, wl_ref, wr_ref, bl_ref,
               g_ref, r_ref):
    x = x_ref[...]
    h = jax.lax.dot_general(x, win_ref[...], (((1,), (1,)), ((), ())),
                            preferred_element_type=jnp.float32)
    h = jnp.maximum(h + bin_ref[...], 0.0)
    g_ref[...] = jax.lax.dot_general(h, wl_ref[...], (((1,), (1,)), ((), ())),
                                     preferred_element_type=jnp.float32)
    r_ref[...] = jax.lax.dot_general(h, wr_ref[...], (((1,), (1,)), ((), ())),
                                     preferred_element_type=jnp.float32) + bl_ref[...]


def _final_body(p0_ref, p1_ref, d0_ref, d1_ref, r_ref, out_ref):
    s = p0_ref[...] + p1_ref[...]
    deg = d0_ref[:, 0:1] + d1_ref[:, 0:1]
    mean = s / jnp.maximum(deg, 1.0)
    o = mean + r_ref[...]
    m = jnp.max(o, axis=1, keepdims=True)
    lse = jnp.log(jnp.sum(jnp.exp(o - m), axis=1, keepdims=True))
    out_ref[...] = o - m - lse


def _sc_edge_agg(g, src, dst, zeros_p, zeros_d):
    mesh = plsc.VectorSubcoreMesh(core_axis_name="c", subcore_axis_name="s")

    @functools.partial(
        pl.kernel,
        out_type=(
            jax.ShapeDtypeStruct((NC * N_PAD, D_OUT), jnp.float32),
            jax.ShapeDtypeStruct((NC * N_PAD, DEG_W), jnp.float32),
        ),
        mesh=mesh,
        compiler_params=pltpu.CompilerParams(use_tc_tiling_on_sc=False),
        scratch_types=[
            pltpu.VMEM((N_CHUNKS, CHUNK), jnp.int32),
            pltpu.VMEM((N_CHUNKS, CHUNK), jnp.int32),
            pltpu.VMEM((3, CHUNK, D_OUT), jnp.float32),
            pltpu.VMEM((CHUNK, DEG_W), jnp.float32),
            pltpu.VMEM_SHARED((N_PAD, D_OUT), jnp.float32),
            pltpu.VMEM_SHARED((N_PAD, DEG_W), jnp.float32),
            pltpu.SemaphoreType.DMA,
            pltpu.SemaphoreType.DMA,
            pltpu.SemaphoreType.DMA,
        ],
    )
    def edge_agg(g_hbm, src_hbm, dst_hbm, zp_hbm, zd_hbm, out_hbm, deg_hbm,
                 src_v, dst_v, rows_v, ones_v, acc_sh, dega_sh,
                 sem0, sem1, sem2):
        c = lax.axis_index("c")
        s = lax.axis_index("s")
        sems = (sem0, sem1, sem2)

        stripe = pl.ds(s * ROWS_PER_TILE, ROWS_PER_TILE)
        pltpu.sync_copy(zp_hbm.at[stripe], acc_sh.at[stripe])
        pltpu.sync_copy(zd_hbm.at[stripe], dega_sh.at[stripe])
        pltpu.sync_copy(src_hbm.at[c, s], src_v)
        pltpu.sync_copy(dst_hbm.at[c, s], dst_v)

        lane = lax.broadcasted_iota(jnp.int32, (DEG_W,), 0)
        one_vec = jnp.where(lane == 0, 1.0, 0.0).astype(jnp.float32)

        def fill(r, carry):
            ones_v[r, :] = one_vec
            return carry

        lax.fori_loop(0, CHUNK, fill, 0)
        plsc.subcore_barrier()

        def gather(i, b):
            pltpu.async_copy(g_hbm.at[src_v.at[i]], rows_v.at[b], sems[b])

        def gwait(i, b):
            pltpu.make_async_copy(g_hbm.at[src_v.at[i]], rows_v.at[b],
                                  sems[b]).wait()

        def step(i, b):
            gwait(i, b)

            @pl.when(i + 2 < N_CHUNKS)
            def _():
                gather(i + 2, (b + 2) % 3)

            pltpu.sync_copy(rows_v.at[b], acc_sh.at[dst_v.at[i]], add=True)
            pltpu.sync_copy(ones_v, dega_sh.at[dst_v.at[i]], add=True)

        gather(0, 0)
        gather(1, 1)

        def body(j, carry):
            i0 = j * 3
            step(i0, 0)
            step(i0 + 1, 1)
            step(i0 + 2, 2)
            return carry

        n_triples = N_CHUNKS // 3
        lax.fori_loop(0, n_triples, body, 0)
        for i in range(n_triples * 3, N_CHUNKS):
            step(i, i % 3)
        plsc.subcore_barrier()

        out_stripe = pl.ds(c * N_PAD + s * ROWS_PER_TILE, ROWS_PER_TILE)
        pltpu.sync_copy(acc_sh.at[stripe], out_hbm.at[out_stripe])
        pltpu.sync_copy(dega_sh.at[stripe], deg_hbm.at[out_stripe])

    return edge_agg(g, src, dst, zeros_p, zeros_d)


def kernel(x, edge_index, W_in, b_in, W_l, b_l, W_r):
    src = edge_index[0].reshape(NC, NS, N_CHUNKS, CHUNK)
    dst = edge_index[1].reshape(NC, NS, N_CHUNKS, CHUNK)
    x_pad = jnp.pad(x, ((0, N_PAD - N_NODES), (0, 0)))
    b_in2 = b_in.reshape(1, D_HID)
    b_l2 = b_l.reshape(1, D_OUT)

    n_blocks = N_PAD // ROW_BLK
    g, r = pl.pallas_call(
        _proj_body,
        grid=(n_blocks,),
        in_specs=[
            pl.BlockSpec((ROW_BLK, D_IN), lambda i: (i, 0)),
            pl.BlockSpec((D_HID, D_IN), lambda i: (0, 0)),
            pl.BlockSpec((1, D_HID), lambda i: (0, 0)),
            pl.BlockSpec((D_OUT, D_HID), lambda i: (0, 0)),
            pl.BlockSpec((D_OUT, D_HID), lambda i: (0, 0)),
            pl.BlockSpec((1, D_OUT), lambda i: (0, 0)),
        ],
        out_specs=[
            pl.BlockSpec((ROW_BLK, D_OUT), lambda i: (i, 0)),
            pl.BlockSpec((ROW_BLK, D_OUT), lambda i: (i, 0)),
        ],
        out_shape=[
            jax.ShapeDtypeStruct((N_PAD, D_OUT), jnp.float32),
            jax.ShapeDtypeStruct((N_PAD, D_OUT), jnp.float32),
        ],
    )(x_pad, W_in, b_in2, W_l, W_r, b_l2)

    zeros_p = jnp.zeros((N_PAD, D_OUT), jnp.float32)
    zeros_d = jnp.zeros((N_PAD, DEG_W), jnp.float32)
    parts, degs = _sc_edge_agg(g, src, dst, zeros_p, zeros_d)

    blocks_per_half = n_blocks
    out = pl.pallas_call(
        _final_body,
        grid=(n_blocks,),
        in_specs=[
            pl.BlockSpec((ROW_BLK, D_OUT), lambda i: (i, 0)),
            pl.BlockSpec((ROW_BLK, D_OUT),
                         lambda i: (i + blocks_per_half, 0)),
            pl.BlockSpec((ROW_BLK, DEG_W), lambda i: (i, 0)),
            pl.BlockSpec((ROW_BLK, DEG_W),
                         lambda i: (i + blocks_per_half, 0)),
            pl.BlockSpec((ROW_BLK, D_OUT), lambda i: (i, 0)),
        ],
        out_specs=pl.BlockSpec((ROW_BLK, D_OUT), lambda i: (i, 0)),
        out_shape=jax.ShapeDtypeStruct((N_PAD, D_OUT), jnp.float32),
    )(parts, parts, degs, degs, r)
    return out[:N_NODES]
